# baseline (speedup 1.0000x reference)
import functools

import jax
import jax.numpy as jnp
from jax import lax
from jax.experimental import pallas as pl
from jax.experimental.pallas import tpu as pltpu
from jax.experimental.pallas import tpu_sc as plsc

N_NODES = 10000
N_EDGES = 320000
D = 128

NC = 2
NS = 16
NW = NC * NS

CHUNK = 128
CHUNKS_PER_W = 80
EPW = CHUNK * CHUNKS_PER_W
REAL_PER_W = N_EDGES // NW
PAD_PER_W = EPW - REAL_PER_W

N_ACC = 11264
ROWS_PER_TILE = N_ACC // NS
GARB_PER_W = (N_ACC - N_NODES) // NS


def _mm_body(f_ref, w_ref, o_ref):
    o_ref[...] = jnp.dot(f_ref[...], w_ref[...],
                         preferred_element_type=jnp.float32)


def _project(feat, W):
    return pl.pallas_call(
        _mm_body,
        grid=(10,),
        in_specs=[pl.BlockSpec((1000, D), lambda i: (i, 0)),
                  pl.BlockSpec((D, D), lambda i: (0, 0))],
        out_specs=pl.BlockSpec((1000, D), lambda i: (i, 0)),
        out_shape=jax.ShapeDtypeStruct((N_NODES, D), jnp.float32),
    )(feat, W)


def _combine_body(p_ref, o_ref):
    o_ref[...] = p_ref[0] + p_ref[1]


def _combine(parts):
    return pl.pallas_call(
        _combine_body,
        grid=(10,),
        in_specs=[pl.BlockSpec((2, 1000, D), lambda i: (0, i, 0))],
        out_specs=pl.BlockSpec((1000, D), lambda i: (i, 0)),
        out_shape=jax.ShapeDtypeStruct((N_NODES, D), jnp.float32),
    )(parts)


def _sc_body(h_hbm, src_hbm, dst_hbm, out_hbm, src_i0, dst_i0, src_i1,
             dst_i1, rows0, rows1, part_sh, sg, ss0, ss1):
    cid = lax.axis_index("c")
    sid = lax.axis_index("s")
    wid = sid * NC + cid

    z = jnp.zeros((16,), jnp.float32)

    def zero_body(r, carry):
        for j in range(D // 16):
            rows0[r, pl.ds(j * 16, 16)] = z
        return carry

    lax.fori_loop(0, CHUNK, zero_body, 0)
    tb = sid * ROWS_PER_TILE
    for r in range(ROWS_PER_TILE // CHUNK):
        pltpu.sync_copy(rows0, part_sh.at[pl.ds(tb + r * CHUNK, CHUNK)])
    rem = ROWS_PER_TILE % CHUNK
    if rem:
        pltpu.sync_copy(
            rows0.at[pl.ds(0, rem)],
            part_sh.at[pl.ds(tb + (ROWS_PER_TILE // CHUNK) * CHUNK, rem)])
    plsc.subcore_barrier()

    def chunk_step(c, src_i, dst_i, rows, ss):
        pltpu.sync_copy(src_hbm.at[wid, c], src_i)
        pltpu.sync_copy(dst_hbm.at[wid, c], dst_i)
        pltpu.async_copy(h_hbm.at[src_i], rows, sg).wait()
        pltpu.async_copy(rows, part_sh.at[dst_i], ss, add=True)

    def scat_wait(rows, dst_i, ss):
        pltpu.make_async_copy(rows, part_sh.at[dst_i], ss).wait()

    chunk_step(0, src_i0, dst_i0, rows0, ss0)
    chunk_step(1, src_i1, dst_i1, rows1, ss1)

    def edge_body(g, carry):
        c0 = 2 + 2 * g
        scat_wait(rows0, dst_i0, ss0)
        chunk_step(c0, src_i0, dst_i0, rows0, ss0)
        scat_wait(rows1, dst_i1, ss1)
        chunk_step(c0 + 1, src_i1, dst_i1, rows1, ss1)
        return carry

    lax.fori_loop(0, (CHUNKS_PER_W - 2) // 2, edge_body, 0)
    scat_wait(rows0, dst_i0, ss0)
    scat_wait(rows1, dst_i1, ss1)

    plsc.subcore_barrier()

    pltpu.sync_copy(part_sh.at[pl.ds(tb, ROWS_PER_TILE)],
                    out_hbm.at[cid, pl.ds(tb, ROWS_PER_TILE)])


_sc_message_passing = functools.partial(
    pl.kernel,
    mesh=plsc.VectorSubcoreMesh(core_axis_name="c", subcore_axis_name="s"),
    out_type=jax.ShapeDtypeStruct((NC, N_ACC, D), jnp.float32),
    scratch_types=[
        pltpu.VMEM((CHUNK,), jnp.int32),
        pltpu.VMEM((CHUNK,), jnp.int32),
        pltpu.VMEM((CHUNK,), jnp.int32),
        pltpu.VMEM((CHUNK,), jnp.int32),
        pltpu.VMEM((CHUNK, D), jnp.float32),
        pltpu.VMEM((CHUNK, D), jnp.float32),
        pltpu.VMEM_SHARED((N_ACC, D), jnp.float32),
        pltpu.SemaphoreType.DMA,
        pltpu.SemaphoreType.DMA,
        pltpu.SemaphoreType.DMA,
    ],
)(_sc_body)


def kernel(feat, edge_index, W):
    src = edge_index[0].astype(jnp.int32).reshape(NW, REAL_PER_W)
    dst = edge_index[1].astype(jnp.int32).reshape(NW, REAL_PER_W)
    w = jnp.arange(NW, dtype=jnp.int32)[:, None]
    pad_src = jnp.zeros((NW, PAD_PER_W), jnp.int32)
    pad_dst = (N_NODES + (w // 2) * GARB_PER_W
               + jnp.arange(PAD_PER_W, dtype=jnp.int32)[None, :] % GARB_PER_W)
    src_p = jnp.concatenate([src, pad_src], axis=1).reshape(
        NW, CHUNKS_PER_W, CHUNK)
    dst_p = jnp.concatenate([dst, pad_dst], axis=1).reshape(
        NW, CHUNKS_PER_W, CHUNK)
    h = _project(feat, W)
    parts = _sc_message_passing(h, src_p, dst_p)
    return _combine(parts)


# speedup vs baseline: 1.3435x; 1.3435x over previous
import functools

import jax
import jax.numpy as jnp
from jax import lax
from jax.experimental import pallas as pl
from jax.experimental.pallas import tpu as pltpu
from jax.experimental.pallas import tpu_sc as plsc

N_NODES = 10000
N_EDGES = 320000
D = 128

NC = 2
NS = 16
NW = NC * NS

CHUNK = 128
CHUNKS_PER_W = 79
EPW = CHUNK * CHUNKS_PER_W
REAL_PER_W = N_EDGES // NW
PAD_PER_W = EPW - REAL_PER_W

N_ACC = 12288
ROWS_PER_TILE = N_ACC // NS
GARB_PER_W = (N_ACC - N_NODES) // NS


def _mm_body(f_ref, w_ref, o_ref):
    o_ref[...] = jnp.dot(f_ref[...], w_ref[...],
                         preferred_element_type=jnp.float32)


def _project(feat, W):
    return pl.pallas_call(
        _mm_body,
        grid=(10,),
        in_specs=[pl.BlockSpec((1000, D), lambda i: (i, 0)),
                  pl.BlockSpec((D, D), lambda i: (0, 0))],
        out_specs=pl.BlockSpec((1000, D), lambda i: (i, 0)),
        out_shape=jax.ShapeDtypeStruct((N_NODES, D), jnp.float32),
    )(feat, W)


def _combine_body(p_ref, o_ref):
    o_ref[...] = p_ref[0] + p_ref[1]


def _combine(parts):
    return pl.pallas_call(
        _combine_body,
        grid=(10,),
        in_specs=[pl.BlockSpec((2, 1000, D), lambda i: (0, i, 0))],
        out_specs=pl.BlockSpec((1000, D), lambda i: (i, 0)),
        out_shape=jax.ShapeDtypeStruct((N_NODES, D), jnp.float32),
    )(parts)


def _sc_body(h_hbm, ei_hbm, out_hbm, idx2, rows0, part_sh, sg0):
    cid = lax.axis_index("c")
    sid = lax.axis_index("s")
    wid = sid * NC + cid

    z = jnp.zeros((16,), jnp.float32)

    def zero_body(r, carry):
        for j in range(D // 16):
            rows0[r, pl.ds(j * 16, 16)] = z
        return carry

    lax.fori_loop(0, CHUNK, zero_body, 0)
    tb = sid * ROWS_PER_TILE
    for r in range(ROWS_PER_TILE // CHUNK):
        pltpu.sync_copy(rows0, part_sh.at[pl.ds(tb + r * CHUNK, CHUNK)])
    plsc.subcore_barrier()

    def edge_body(c, carry):
        pltpu.sync_copy(ei_hbm.at[wid, c], idx2)
        pltpu.async_copy(h_hbm.at[idx2.at[0]], rows0, sg0).wait()
        pltpu.sync_copy(rows0, part_sh.at[idx2.at[1]], add=True)
        return carry

    lax.fori_loop(0, CHUNKS_PER_W, edge_body, 0)
    plsc.subcore_barrier()

    pltpu.sync_copy(part_sh.at[pl.ds(tb, ROWS_PER_TILE)],
                    out_hbm.at[cid, pl.ds(tb, ROWS_PER_TILE)])


_sc_message_passing = functools.partial(
    pl.kernel,
    mesh=plsc.VectorSubcoreMesh(core_axis_name="c", subcore_axis_name="s"),
    out_type=jax.ShapeDtypeStruct((NC, N_ACC, D), jnp.float32),
    scratch_types=[
        pltpu.VMEM((2, CHUNK), jnp.int32),
        pltpu.VMEM((CHUNK, D), jnp.float32),
        pltpu.VMEM_SHARED((N_ACC, D), jnp.float32),
        pltpu.SemaphoreType.DMA,
    ],
)(_sc_body)


def kernel(feat, edge_index, W):
    src = edge_index[0].astype(jnp.int32).reshape(NW, REAL_PER_W)
    dst = edge_index[1].astype(jnp.int32).reshape(NW, REAL_PER_W)
    w = jnp.arange(NW, dtype=jnp.int32)[:, None]
    pad_src = jnp.zeros((NW, PAD_PER_W), jnp.int32)
    pad_dst = (N_NODES + (w // 2) * GARB_PER_W
               + jnp.arange(PAD_PER_W, dtype=jnp.int32)[None, :] % GARB_PER_W)
    src_p = jnp.concatenate([src, pad_src], axis=1).reshape(
        NW, CHUNKS_PER_W, CHUNK)
    dst_p = jnp.concatenate([dst, pad_dst], axis=1).reshape(
        NW, CHUNKS_PER_W, CHUNK)
    ei = jnp.stack([src_p, dst_p], axis=2)
    h = _project(feat, W)
    parts = _sc_message_passing(h, ei)
    return _combine(parts)


# speedup vs baseline: 1.4715x; 1.0953x over previous
import functools

import jax
import jax.numpy as jnp
from jax import lax
from jax.experimental import pallas as pl
from jax.experimental.pallas import tpu as pltpu
from jax.experimental.pallas import tpu_sc as plsc

N_NODES = 10000
N_EDGES = 320000
D = 128

NC = 2
NS = 16
NW = NC * NS

CHUNK = 128
CHUNKS_PER_W = 79
EPW = CHUNK * CHUNKS_PER_W
REAL_PER_W = N_EDGES // NW
PAD_PER_W = EPW - REAL_PER_W

N_ACC = 12288
ROWS_PER_TILE = N_ACC // NS
GARB_PER_W = (N_ACC - N_NODES) // NS


def _mm_body(f_ref, w_ref, o_ref):
    o_ref[...] = jnp.dot(f_ref[...], w_ref[...],
                         preferred_element_type=jnp.float32)


def _project(feat, W):
    return pl.pallas_call(
        _mm_body,
        grid=(10,),
        in_specs=[pl.BlockSpec((1000, D), lambda i: (i, 0)),
                  pl.BlockSpec((D, D), lambda i: (0, 0))],
        out_specs=pl.BlockSpec((1000, D), lambda i: (i, 0)),
        out_shape=jax.ShapeDtypeStruct((N_NODES, D), jnp.float32),
    )(feat, W)


def _combine_body(p_ref, o_ref):
    o_ref[...] = p_ref[0] + p_ref[1]


def _combine(parts):
    return pl.pallas_call(
        _combine_body,
        grid=(10,),
        in_specs=[pl.BlockSpec((2, 1000, D), lambda i: (0, i, 0))],
        out_specs=pl.BlockSpec((1000, D), lambda i: (i, 0)),
        out_shape=jax.ShapeDtypeStruct((N_NODES, D), jnp.float32),
    )(parts)


def _sc_body(h_hbm, ei_hbm, out_hbm, idx2a, idx2b, rows0, part_sh, sg0,
             si):
    cid = lax.axis_index("c")
    sid = lax.axis_index("s")
    wid = sid * NC + cid

    z = jnp.zeros((16,), jnp.float32)

    def zero_body(r, carry):
        for j in range(D // 16):
            rows0[r, pl.ds(j * 16, 16)] = z
        return carry

    lax.fori_loop(0, CHUNK, zero_body, 0)
    tb = sid * ROWS_PER_TILE
    for r in range(ROWS_PER_TILE // CHUNK):
        pltpu.sync_copy(rows0, part_sh.at[pl.ds(tb + r * CHUNK, CHUNK)])
    plsc.subcore_barrier()

    def gs(idx2):
        pltpu.async_copy(h_hbm.at[idx2.at[0]], rows0, sg0).wait()
        pltpu.sync_copy(rows0, part_sh.at[idx2.at[1]], add=True)

    pltpu.sync_copy(ei_hbm.at[wid, 0], idx2a)

    def edge_body(g, carry):
        c0 = 2 * g
        pltpu.async_copy(ei_hbm.at[wid, c0 + 1], idx2b, si)
        gs(idx2a)
        pltpu.make_async_copy(ei_hbm.at[wid, 0], idx2b, si).wait()
        pltpu.async_copy(ei_hbm.at[wid, c0 + 2], idx2a, si)
        gs(idx2b)
        pltpu.make_async_copy(ei_hbm.at[wid, 0], idx2a, si).wait()
        return carry

    lax.fori_loop(0, (CHUNKS_PER_W - 1) // 2, edge_body, 0)
    gs(idx2a)
    plsc.subcore_barrier()

    pltpu.sync_copy(part_sh.at[pl.ds(tb, ROWS_PER_TILE)],
                    out_hbm.at[cid, pl.ds(tb, ROWS_PER_TILE)])


_sc_message_passing = functools.partial(
    pl.kernel,
    mesh=plsc.VectorSubcoreMesh(core_axis_name="c", subcore_axis_name="s"),
    out_type=jax.ShapeDtypeStruct((NC, N_ACC, D), jnp.float32),
    scratch_types=[
        pltpu.VMEM((2, CHUNK), jnp.int32),
        pltpu.VMEM((2, CHUNK), jnp.int32),
        pltpu.VMEM((CHUNK, D), jnp.float32),
        pltpu.VMEM_SHARED((N_ACC, D), jnp.float32),
        pltpu.SemaphoreType.DMA,
        pltpu.SemaphoreType.DMA,
    ],
)(_sc_body)


def kernel(feat, edge_index, W):
    src = edge_index[0].astype(jnp.int32).reshape(NW, REAL_PER_W)
    dst = edge_index[1].astype(jnp.int32).reshape(NW, REAL_PER_W)
    w = jnp.arange(NW, dtype=jnp.int32)[:, None]
    pad_src = jnp.zeros((NW, PAD_PER_W), jnp.int32)
    pad_dst = (N_NODES + (w // 2) * GARB_PER_W
               + jnp.arange(PAD_PER_W, dtype=jnp.int32)[None, :] % GARB_PER_W)
    src_p = jnp.concatenate([src, pad_src], axis=1).reshape(
        NW, CHUNKS_PER_W, CHUNK)
    dst_p = jnp.concatenate([dst, pad_dst], axis=1).reshape(
        NW, CHUNKS_PER_W, CHUNK)
    ei = jnp.stack([src_p, dst_p], axis=2)
    h = _project(feat, W)
    parts = _sc_message_passing(h, ei)
    return _combine(parts)
